# Initial kernel scaffold; baseline (speedup 1.0000x reference)
import functools

import jax
import jax.numpy as jnp
from jax import lax
from jax.experimental import pallas as pl
from jax.experimental.pallas import tpu as pltpu
from jax.experimental.pallas import tpu_sc as plsc

_NC = 2
_NS = 16
_L = 16

_ROWS = 2
_E = 1_600_000
_DIL = 2
_OUT_E = _E // _DIL

_SEGS_PER_ROW = (_NC * _NS) // _ROWS
_IN_W = _E // _SEGS_PER_ROW
_OUT_W = _IN_W // _DIL

_N_CHUNK = 5
_IN_CHUNK = _IN_W // _N_CHUNK
_OUT_CHUNK = _IN_CHUNK // _DIL

_mesh = plsc.VectorSubcoreMesh(
    core_axis_name="c", subcore_axis_name="s", num_cores=_NC, num_subcores=_NS
)


@functools.partial(
    pl.kernel,
    out_type=jax.ShapeDtypeStruct((_ROWS * _OUT_E,), jnp.int32),
    mesh=_mesh,
    scratch_types=[
        pltpu.VMEM((_IN_CHUNK,), jnp.int32),
        pltpu.VMEM((_OUT_CHUNK,), jnp.int32),
    ],
)
def _dilate(in_hbm, out_hbm, in_v, out_v):
    cid = lax.axis_index("c")
    sid = lax.axis_index("s")
    wid = sid * _NC + cid
    row = wid // _SEGS_PER_ROW
    seg = wid % _SEGS_PER_ROW
    in_base = row * _E + seg * _IN_W
    out_base = row * _OUT_E + seg * _OUT_W

    lane = lax.iota(jnp.int32, _L)

    for k in range(_N_CHUNK):
        pltpu.sync_copy(in_hbm.at[pl.ds(in_base + k * _IN_CHUNK, _IN_CHUNK)], in_v)

        def inner(i, carry):
            idx = (lane + i * _L) * _DIL
            val = plsc.load_gather(in_v, [idx])
            out_v[pl.ds(i * _L, _L)] = val
            return carry

        lax.fori_loop(0, _OUT_CHUNK // _L, inner, 0)

        pltpu.sync_copy(out_v, out_hbm.at[pl.ds(out_base + k * _OUT_CHUNK, _OUT_CHUNK)])


def kernel(edge_index, edge_attr):
    out_flat = _dilate(edge_index.reshape(-1))
    return out_flat.reshape(_ROWS, _OUT_E), edge_attr


# speedup vs baseline: 2.9217x; 2.9217x over previous
import functools

import jax
import jax.numpy as jnp
from jax import lax
from jax.experimental import pallas as pl
from jax.experimental.pallas import tpu as pltpu
from jax.experimental.pallas import tpu_sc as plsc

_NC = 2
_NS = 16
_L = 16

_ROWS = 2
_E = 1_600_000
_DIL = 2
_OUT_E = _E // _DIL

_SEGS_PER_ROW = (_NC * _NS) // _ROWS
_IN_W = _E // _SEGS_PER_ROW
_OUT_W = _IN_W // _DIL

_N_CHUNK = 5
_IN_CHUNK = _IN_W // _N_CHUNK
_OUT_CHUNK = _IN_CHUNK // _DIL

_mesh = plsc.VectorSubcoreMesh(
    core_axis_name="c", subcore_axis_name="s", num_cores=_NC, num_subcores=_NS
)


@functools.partial(
    pl.kernel,
    out_type=jax.ShapeDtypeStruct((_ROWS * _OUT_E,), jnp.int32),
    mesh=_mesh,
    scratch_types=[
        pltpu.VMEM((_IN_CHUNK,), jnp.int32),
        pltpu.VMEM((_OUT_CHUNK,), jnp.int32),
    ],
    compiler_params=pltpu.CompilerParams(needs_layout_passes=False),
)
def _dilate(in_hbm, out_hbm, in_v, out_v):
    cid = lax.axis_index("c")
    sid = lax.axis_index("s")
    wid = sid * _NC + cid
    row = wid // _SEGS_PER_ROW
    seg = wid % _SEGS_PER_ROW
    in_base = row * _E + seg * _IN_W
    out_base = row * _OUT_E + seg * _OUT_W

    lane = lax.iota(jnp.int32, _L)

    for k in range(_N_CHUNK):
        pltpu.sync_copy(in_hbm.at[pl.ds(in_base + k * _IN_CHUNK, _IN_CHUNK)], in_v)

        def inner(i, carry):
            idx = (lane + i * _L) * _DIL
            val = plsc.load_gather(in_v, [idx])
            out_v[pl.ds(i * _L, _L)] = val
            return carry

        lax.fori_loop(0, _OUT_CHUNK // _L, inner, 0)

        pltpu.sync_copy(out_v, out_hbm.at[pl.ds(out_base + k * _OUT_CHUNK, _OUT_CHUNK)])


def kernel(edge_index, edge_attr):
    out_flat = _dilate(edge_index.reshape(-1))
    return out_flat.reshape(_ROWS, _OUT_E), edge_attr


# speedup vs baseline: 3.4814x; 1.1916x over previous
import functools

import jax
import jax.numpy as jnp
from jax import lax
from jax.experimental import pallas as pl
from jax.experimental.pallas import tpu as pltpu
from jax.experimental.pallas import tpu_sc as plsc

_NC = 2
_NS = 16
_L = 16

_ROWS = 2
_E = 1_600_000
_DIL = 2
_OUT_E = _E // _DIL

_SEGS_PER_ROW = (_NC * _NS) // _ROWS
_IN_W = _E // _SEGS_PER_ROW
_OUT_W = _IN_W // _DIL

_N_CHUNK = 5
_IN_CHUNK = _IN_W // _N_CHUNK
_OUT_CHUNK = _IN_CHUNK // _DIL

_mesh = plsc.VectorSubcoreMesh(
    core_axis_name="c", subcore_axis_name="s", num_cores=_NC, num_subcores=_NS
)


@functools.partial(
    pl.kernel,
    out_type=jax.ShapeDtypeStruct((_ROWS * _OUT_E,), jnp.int32),
    mesh=_mesh,
    scratch_types=[
        pltpu.VMEM((_IN_CHUNK,), jnp.int32),
        pltpu.VMEM((_IN_CHUNK,), jnp.int32),
        pltpu.VMEM((_OUT_CHUNK,), jnp.int32),
        pltpu.VMEM((_OUT_CHUNK,), jnp.int32),
        pltpu.SemaphoreType.DMA,
        pltpu.SemaphoreType.DMA,
        pltpu.SemaphoreType.DMA,
        pltpu.SemaphoreType.DMA,
    ],
    compiler_params=pltpu.CompilerParams(needs_layout_passes=False),
)
def _dilate(in_hbm, out_hbm, in_v0, in_v1, out_v0, out_v1, sin0, sin1, sout0, sout1):
    cid = lax.axis_index("c")
    sid = lax.axis_index("s")
    wid = sid * _NC + cid
    row = wid // _SEGS_PER_ROW
    seg = wid % _SEGS_PER_ROW
    in_base = row * _E + seg * _IN_W
    out_base = row * _OUT_E + seg * _OUT_W

    lane = lax.iota(jnp.int32, _L)
    in_bufs = (in_v0, in_v1)
    out_bufs = (out_v0, out_v1)
    sins = (sin0, sin1)
    souts = (sout0, sout1)

    in_dma = [None] * _N_CHUNK
    out_dma = [None] * _N_CHUNK

    def start_in(k):
        in_dma[k] = pltpu.async_copy(
            in_hbm.at[pl.ds(in_base + k * _IN_CHUNK, _IN_CHUNK)],
            in_bufs[k % 2], sins[k % 2])

    start_in(0)
    for k in range(_N_CHUNK):
        if k + 1 < _N_CHUNK:
            start_in(k + 1)
        in_dma[k].wait()
        if k >= 2:
            out_dma[k - 2].wait()

        src = in_bufs[k % 2]
        dst = out_bufs[k % 2]

        @plsc.parallel_loop(0, _OUT_CHUNK // _L, unroll=8)
        def _(i):
            idx = (lane + i * _L) * _DIL
            dst[pl.ds(i * _L, _L)] = plsc.load_gather(src, [idx])

        out_dma[k] = pltpu.async_copy(
            dst, out_hbm.at[pl.ds(out_base + k * _OUT_CHUNK, _OUT_CHUNK)],
            souts[k % 2])

    out_dma[_N_CHUNK - 2].wait()
    out_dma[_N_CHUNK - 1].wait()


def kernel(edge_index, edge_attr):
    out_flat = _dilate(edge_index.reshape(-1))
    return out_flat.reshape(_ROWS, _OUT_E), edge_attr


# speedup vs baseline: 6.7451x; 1.9375x over previous
import functools

import jax
import jax.numpy as jnp
from jax import lax
from jax.experimental import pallas as pl
from jax.experimental.pallas import tpu as pltpu
from jax.experimental.pallas import tpu_sc as plsc

_NC = 2
_NS = 16
_NW = _NC * _NS
_L = 16

_ROWS = 2
_E = 1_600_000
_DIL = 2
_OUT_E = _E // _DIL

_CC = 12_800
_OC = _CC // _DIL
_N_CHUNK = _E // _CC
_ROUNDS = -(-_N_CHUNK // _NW)

_mesh = plsc.VectorSubcoreMesh(
    core_axis_name="c", subcore_axis_name="s", num_cores=_NC, num_subcores=_NS
)


@functools.partial(
    pl.kernel,
    out_type=jax.ShapeDtypeStruct((_ROWS, _OUT_E), jnp.int32),
    mesh=_mesh,
    scratch_types=[
        pltpu.VMEM((_ROWS, _CC), jnp.int32),
        pltpu.VMEM((_ROWS, _CC), jnp.int32),
        pltpu.VMEM((_ROWS, _OC), jnp.int32),
        pltpu.VMEM((_ROWS, _OC), jnp.int32),
        pltpu.SemaphoreType.DMA,
        pltpu.SemaphoreType.DMA,
        pltpu.SemaphoreType.DMA,
        pltpu.SemaphoreType.DMA,
    ],
    compiler_params=pltpu.CompilerParams(needs_layout_passes=False),
)
def _dilate(in_hbm, out_hbm, in_v0, in_v1, out_v0, out_v1, sin0, sin1, sout0, sout1):
    cid = lax.axis_index("c")
    sid = lax.axis_index("s")
    wid = sid * _NC + cid

    lane = lax.iota(jnp.int32, _L)
    in_bufs = (in_v0, in_v1)
    out_bufs = (out_v0, out_v1)
    sins = (sin0, sin1)
    souts = (sout0, sout1)

    def chunk_id(t):
        return wid + t * _NW

    def valid(t):
        return chunk_id(t) < _N_CHUNK

    def in_pair(t):
        b = t % 2
        return (in_hbm.at[:, pl.ds(chunk_id(t) * _CC, _CC)], in_bufs[b], sins[b])

    def out_pair(t):
        b = t % 2
        return (out_bufs[b], out_hbm.at[:, pl.ds(chunk_id(t) * _OC, _OC)], souts[b])

    @pl.when(valid(0))
    def _():
        pltpu.async_copy(*in_pair(0))

    for t in range(_ROUNDS):
        if t + 1 < _ROUNDS:
            @pl.when(valid(t + 1))
            def _(t=t):
                pltpu.async_copy(*in_pair(t + 1))

        @pl.when(valid(t))
        def _(t=t):
            pltpu.make_async_copy(*in_pair(t)).wait()
            if t >= 2:
                pltpu.make_async_copy(*out_pair(t - 2)).wait()
            src = in_bufs[t % 2]
            dst = out_bufs[t % 2]
            for r in range(_ROWS):
                row_idx = jnp.full((_L,), r, jnp.int32)

                @plsc.parallel_loop(0, _OC // _L, unroll=8)
                def _(i):
                    col_idx = (lane + i * _L) * _DIL
                    dst[r, pl.ds(i * _L, _L)] = plsc.load_gather(
                        src, [row_idx, col_idx])

            pltpu.async_copy(*out_pair(t))

    for t in (_ROUNDS - 2, _ROUNDS - 1):
        @pl.when(valid(t))
        def _(t=t):
            pltpu.make_async_copy(*out_pair(t)).wait()


def kernel(edge_index, edge_attr):
    return _dilate(edge_index), edge_attr


# speedup vs baseline: 7.1647x; 1.0622x over previous
import functools

import jax
import jax.numpy as jnp
from jax import lax
from jax.experimental import pallas as pl
from jax.experimental.pallas import tpu as pltpu
from jax.experimental.pallas import tpu_sc as plsc

_NC = 2
_NS = 16
_NW = _NC * _NS
_L = 16

_ROWS = 2
_E = 1_600_000
_DIL = 2
_OUT_E = _E // _DIL

_CC = 12_800
_OC = _CC // _DIL
_N_CHUNK = _E // _CC
_ROUNDS = -(-_N_CHUNK // _NW)

_mesh = plsc.VectorSubcoreMesh(
    core_axis_name="c", subcore_axis_name="s", num_cores=_NC, num_subcores=_NS
)


@functools.partial(
    pl.kernel,
    out_type=(
        jax.ShapeDtypeStruct((_ROWS, _OUT_E), jnp.int32),
        jax.ShapeDtypeStruct((1, _E), jnp.float32),
    ),
    mesh=_mesh,
    scratch_types=[
        pltpu.VMEM((_ROWS, _CC), jnp.int32),
        pltpu.VMEM((_ROWS, _CC), jnp.int32),
        pltpu.VMEM((_ROWS, _OC), jnp.int32),
        pltpu.VMEM((_ROWS, _OC), jnp.int32),
        pltpu.VMEM((1, _CC), jnp.float32),
        pltpu.VMEM((1, _CC), jnp.float32),
        pltpu.SemaphoreType.DMA,
        pltpu.SemaphoreType.DMA,
        pltpu.SemaphoreType.DMA,
        pltpu.SemaphoreType.DMA,
        pltpu.SemaphoreType.DMA,
        pltpu.SemaphoreType.DMA,
        pltpu.SemaphoreType.DMA,
        pltpu.SemaphoreType.DMA,
    ],
    compiler_params=pltpu.CompilerParams(needs_layout_passes=False),
)
def _dilate(in_hbm, attr_hbm, out_hbm, attr_out_hbm,
            in_v0, in_v1, out_v0, out_v1, at_v0, at_v1,
            sin0, sin1, sout0, sout1, sai0, sai1, sao0, sao1):
    cid = lax.axis_index("c")
    sid = lax.axis_index("s")
    wid = sid * _NC + cid

    lane = lax.iota(jnp.int32, _L)
    in_bufs = (in_v0, in_v1)
    out_bufs = (out_v0, out_v1)
    at_bufs = (at_v0, at_v1)
    sins = (sin0, sin1)
    souts = (sout0, sout1)
    sais = (sai0, sai1)
    saos = (sao0, sao1)

    def chunk_id(t):
        return wid + t * _NW

    def valid(t):
        return chunk_id(t) < _N_CHUNK

    def in_pair(t):
        b = t % 2
        return (in_hbm.at[:, pl.ds(chunk_id(t) * _CC, _CC)], in_bufs[b], sins[b])

    def out_pair(t):
        b = t % 2
        return (out_bufs[b], out_hbm.at[:, pl.ds(chunk_id(t) * _OC, _OC)], souts[b])

    def attr_in_pair(t):
        b = t % 2
        return (attr_hbm.at[:, pl.ds(chunk_id(t) * _CC, _CC)], at_bufs[b], sais[b])

    def attr_out_pair(t):
        b = t % 2
        return (at_bufs[b], attr_out_hbm.at[:, pl.ds(chunk_id(t) * _CC, _CC)], saos[b])

    @pl.when(valid(0))
    def _():
        pltpu.async_copy(*in_pair(0))
        pltpu.async_copy(*attr_in_pair(0))

    for t in range(_ROUNDS):
        if t + 1 < _ROUNDS:
            @pl.when(valid(t + 1))
            def _(t=t):
                pltpu.async_copy(*in_pair(t + 1))
                pltpu.async_copy(*attr_in_pair(t + 1))

        @pl.when(valid(t))
        def _(t=t):
            if t >= 2:
                pltpu.make_async_copy(*out_pair(t - 2)).wait()
                pltpu.make_async_copy(*attr_out_pair(t - 2)).wait()
            pltpu.make_async_copy(*attr_in_pair(t)).wait()
            pltpu.async_copy(*attr_out_pair(t))
            pltpu.make_async_copy(*in_pair(t)).wait()
            src = in_bufs[t % 2]
            dst = out_bufs[t % 2]
            for r in range(_ROWS):
                row_idx = jnp.full((_L,), r, jnp.int32)

                @plsc.parallel_loop(0, _OC // _L, unroll=8)
                def _(i):
                    col_idx = (lane + i * _L) * _DIL
                    dst[r, pl.ds(i * _L, _L)] = plsc.load_gather(
                        src, [row_idx, col_idx])

            pltpu.async_copy(*out_pair(t))

    for t in (_ROUNDS - 2, _ROUNDS - 1):
        @pl.when(valid(t))
        def _(t=t):
            pltpu.make_async_copy(*out_pair(t)).wait()
            pltpu.make_async_copy(*attr_out_pair(t)).wait()


def kernel(edge_index, edge_attr):
    out, attr_out = _dilate(edge_index, edge_attr)
    return out, attr_out


# speedup vs baseline: 7.2125x; 1.0067x over previous
import functools

import jax
import jax.numpy as jnp
from jax import lax
from jax.experimental import pallas as pl
from jax.experimental.pallas import tpu as pltpu
from jax.experimental.pallas import tpu_sc as plsc

_NC = 2
_NS = 16
_NW = _NC * _NS
_L = 16

_ROWS = 2
_E = 1_600_000
_DIL = 2
_OUT_E = _E // _DIL

_CC = 12_800
_OC = _CC // _DIL
_N_CHUNK = _E // _CC
_ROUNDS = -(-_N_CHUNK // _NW)

_mesh = plsc.VectorSubcoreMesh(
    core_axis_name="c", subcore_axis_name="s", num_cores=_NC, num_subcores=_NS
)


@functools.partial(
    pl.kernel,
    out_type=(
        jax.ShapeDtypeStruct((_ROWS, _OUT_E), jnp.int32),
        jax.ShapeDtypeStruct((1, _E), jnp.float32),
    ),
    mesh=_mesh,
    scratch_types=[
        pltpu.VMEM((_ROWS, _CC), jnp.int32),
        pltpu.VMEM((_ROWS, _CC), jnp.int32),
        pltpu.VMEM((_ROWS, _OC), jnp.int32),
        pltpu.VMEM((_ROWS, _OC), jnp.int32),
        pltpu.VMEM((1, _CC), jnp.float32),
        pltpu.VMEM((1, _CC), jnp.float32),
        pltpu.SemaphoreType.DMA,
        pltpu.SemaphoreType.DMA,
        pltpu.SemaphoreType.DMA,
        pltpu.SemaphoreType.DMA,
        pltpu.SemaphoreType.DMA,
        pltpu.SemaphoreType.DMA,
        pltpu.SemaphoreType.DMA,
        pltpu.SemaphoreType.DMA,
    ],
    compiler_params=pltpu.CompilerParams(needs_layout_passes=False),
)
def _dilate(in_hbm, attr_hbm, out_hbm, attr_out_hbm,
            in_v0, in_v1, out_v0, out_v1, at_v0, at_v1,
            sin0, sin1, sout0, sout1, sai0, sai1, sao0, sao1):
    cid = lax.axis_index("c")
    sid = lax.axis_index("s")
    wid = sid * _NC + cid

    lane = lax.iota(jnp.int32, _L)
    in_bufs = (in_v0, in_v1)
    out_bufs = (out_v0, out_v1)
    at_bufs = (at_v0, at_v1)
    sins = (sin0, sin1)
    souts = (sout0, sout1)
    sais = (sai0, sai1)
    saos = (sao0, sao1)

    def chunk_id(t):
        return wid + t * _NW

    def valid(t):
        return chunk_id(t) < _N_CHUNK

    def in_pair(t, b):
        return (in_hbm.at[:, pl.ds(chunk_id(t) * _CC, _CC)], in_bufs[b], sins[b])

    def out_pair(t, b):
        return (out_bufs[b], out_hbm.at[:, pl.ds(chunk_id(t) * _OC, _OC)], souts[b])

    def attr_in_pair(t, b):
        return (attr_hbm.at[:, pl.ds(chunk_id(t) * _CC, _CC)], at_bufs[b], sais[b])

    def attr_out_pair(t, b):
        return (at_bufs[b], attr_out_hbm.at[:, pl.ds(chunk_id(t) * _CC, _CC)], saos[b])

    @pl.when(valid(0))
    def _():
        pltpu.async_copy(*in_pair(0, 0))
        pltpu.async_copy(*attr_in_pair(0, 0))

    def round_body(t, b):
        @pl.when(valid(t + 1))
        def _():
            @pl.when(t >= 1)
            def _():
                pltpu.make_async_copy(*out_pair(t - 1, 1 - b)).wait()
                pltpu.make_async_copy(*attr_out_pair(t - 1, 1 - b)).wait()
            pltpu.async_copy(*in_pair(t + 1, 1 - b))
            pltpu.async_copy(*attr_in_pair(t + 1, 1 - b))

        @pl.when(valid(t))
        def _():
            pltpu.make_async_copy(*attr_in_pair(t, b)).wait()
            pltpu.async_copy(*attr_out_pair(t, b))
            pltpu.make_async_copy(*in_pair(t, b)).wait()
            src = in_bufs[b]
            dst = out_bufs[b]
            for r in range(_ROWS):
                row_idx = jnp.full((_L,), r, jnp.int32)

                @plsc.parallel_loop(0, _OC // _L, unroll=4)
                def _(i):
                    col_idx = (lane + i * _L) * _DIL
                    dst[r, pl.ds(i * _L, _L)] = plsc.load_gather(
                        src, [row_idx, col_idx])

            pltpu.async_copy(*out_pair(t, b))

    def pair_body(tt, carry):
        round_body(2 * tt, 0)
        round_body(2 * tt + 1, 1)
        return carry

    lax.fori_loop(0, _ROUNDS // 2, pair_body, 0)

    for t in range(_ROUNDS):
        @pl.when(jnp.logical_and(valid(t), jnp.logical_not(valid(t + 2))))
        def _(t=t):
            pltpu.make_async_copy(*out_pair(t, t % 2)).wait()
            pltpu.make_async_copy(*attr_out_pair(t, t % 2)).wait()


def kernel(edge_index, edge_attr):
    out, attr_out = _dilate(edge_index, edge_attr)
    return out, attr_out


# speedup vs baseline: 7.3891x; 1.0245x over previous
import functools

import jax
import jax.numpy as jnp
from jax import lax
from jax.experimental import pallas as pl
from jax.experimental.pallas import tpu as pltpu
from jax.experimental.pallas import tpu_sc as plsc

_NC = 2
_NS = 16
_NW = _NC * _NS
_L = 16

_ROWS = 2
_E = 1_600_000
_DIL = 2
_OUT_E = _E // _DIL

_CC = 6_400
_OC = _CC // _DIL
_N_CHUNK = _E // _CC
_ROUNDS = -(-_N_CHUNK // _NW)
_NBUF = 4
_PD = 2

_mesh = plsc.VectorSubcoreMesh(
    core_axis_name="c", subcore_axis_name="s", num_cores=_NC, num_subcores=_NS
)


@functools.partial(
    pl.kernel,
    out_type=(
        jax.ShapeDtypeStruct((_ROWS, _OUT_E), jnp.int32),
        jax.ShapeDtypeStruct((1, _E), jnp.float32),
    ),
    mesh=_mesh,
    scratch_types=(
        [pltpu.VMEM((_ROWS, _CC), jnp.int32) for _ in range(_NBUF)]
        + [pltpu.VMEM((_ROWS, _OC), jnp.int32) for _ in range(_NBUF)]
        + [pltpu.VMEM((1, _CC), jnp.float32) for _ in range(_NBUF)]
        + [pltpu.SemaphoreType.DMA for _ in range(4 * _NBUF)]
    ),
    compiler_params=pltpu.CompilerParams(needs_layout_passes=False),
)
def _dilate(in_hbm, attr_hbm, out_hbm, attr_out_hbm, *bufs_and_sems):
    in_bufs = bufs_and_sems[:_NBUF]
    out_bufs = bufs_and_sems[_NBUF:2 * _NBUF]
    at_bufs = bufs_and_sems[2 * _NBUF:3 * _NBUF]
    sems = bufs_and_sems[3 * _NBUF:]
    sins = sems[:_NBUF]
    souts = sems[_NBUF:2 * _NBUF]
    sais = sems[2 * _NBUF:3 * _NBUF]
    saos = sems[3 * _NBUF:]

    cid = lax.axis_index("c")
    sid = lax.axis_index("s")
    wid = sid * _NC + cid

    lane = lax.iota(jnp.int32, _L)

    def chunk_id(t):
        return wid + t * _NW

    def valid(t):
        return chunk_id(t) < _N_CHUNK

    def in_pair(t, b):
        return (in_hbm.at[:, pl.ds(chunk_id(t) * _CC, _CC)], in_bufs[b], sins[b])

    def out_pair(t, b):
        return (out_bufs[b], out_hbm.at[:, pl.ds(chunk_id(t) * _OC, _OC)], souts[b])

    def attr_in_pair(t, b):
        return (attr_hbm.at[:, pl.ds(chunk_id(t) * _CC, _CC)], at_bufs[b], sais[b])

    def attr_out_pair(t, b):
        return (at_bufs[b], attr_out_hbm.at[:, pl.ds(chunk_id(t) * _CC, _CC)], saos[b])

    for p in range(_PD):
        @pl.when(valid(p))
        def _(p=p):
            pltpu.async_copy(*in_pair(p, p % _NBUF))
            pltpu.async_copy(*attr_in_pair(p, p % _NBUF))

    def round_body(t, b):
        bp = (b + _PD) % _NBUF

        @pl.when(valid(t + _PD))
        def _():
            @pl.when(t + _PD >= _NBUF)
            def _():
                pltpu.make_async_copy(*out_pair(t + _PD - _NBUF, bp)).wait()
                pltpu.make_async_copy(*attr_out_pair(t + _PD - _NBUF, bp)).wait()
            pltpu.async_copy(*in_pair(t + _PD, bp))
            pltpu.async_copy(*attr_in_pair(t + _PD, bp))

        @pl.when(valid(t))
        def _():
            pltpu.make_async_copy(*attr_in_pair(t, b)).wait()
            pltpu.async_copy(*attr_out_pair(t, b))
            pltpu.make_async_copy(*in_pair(t, b)).wait()
            src = in_bufs[b]
            dst = out_bufs[b]
            for r in range(_ROWS):
                row_idx = jnp.full((_L,), r, jnp.int32)

                @plsc.parallel_loop(0, _OC // _L, unroll=4)
                def _(i):
                    col_idx = (lane + i * _L) * _DIL
                    dst[r, pl.ds(i * _L, _L)] = plsc.load_gather(
                        src, [row_idx, col_idx])

            pltpu.async_copy(*out_pair(t, b))

    def ring_body(tt, carry):
        for b in range(_NBUF):
            round_body(_NBUF * tt + b, b)
        return carry

    lax.fori_loop(0, _ROUNDS // _NBUF, ring_body, 0)

    for t in range(_ROUNDS):
        @pl.when(jnp.logical_and(valid(t), jnp.logical_not(valid(t + _NBUF))))
        def _(t=t):
            pltpu.make_async_copy(*out_pair(t, t % _NBUF)).wait()
            pltpu.make_async_copy(*attr_out_pair(t, t % _NBUF)).wait()


def kernel(edge_index, edge_attr):
    out, attr_out = _dilate(edge_index, edge_attr)
    return out, attr_out


# speedup vs baseline: 7.3979x; 1.0012x over previous
import functools

import jax
import jax.numpy as jnp
from jax import lax
from jax.experimental import pallas as pl
from jax.experimental.pallas import tpu as pltpu
from jax.experimental.pallas import tpu_sc as plsc

_NC = 2
_NS = 16
_NW = _NC * _NS
_L = 16

_ROWS = 2
_E = 1_600_000
_DIL = 2
_OUT_E = _E // _DIL

_CC = 6_400
_OC = _CC // _DIL
_N_CHUNK = _E // _CC
_ROUNDS = -(-_N_CHUNK // _NW)
_NBUF = 4
_PD = 2

_mesh = plsc.VectorSubcoreMesh(
    core_axis_name="c", subcore_axis_name="s", num_cores=_NC, num_subcores=_NS
)


@functools.partial(
    pl.kernel,
    out_type=(
        jax.ShapeDtypeStruct((_ROWS, _OUT_E), jnp.int32),
        jax.ShapeDtypeStruct((1, _E), jnp.float32),
    ),
    mesh=_mesh,
    scratch_types=(
        [pltpu.VMEM((_ROWS, _CC), jnp.int32) for _ in range(_NBUF)]
        + [pltpu.VMEM((_ROWS, _OC), jnp.int32) for _ in range(_NBUF)]
        + [pltpu.VMEM((1, _CC), jnp.float32) for _ in range(_NBUF)]
        + [pltpu.SemaphoreType.DMA for _ in range(4 * _NBUF)]
    ),
    compiler_params=pltpu.CompilerParams(
        needs_layout_passes=False, skip_device_barrier=True),
)
def _dilate(in_hbm, attr_hbm, out_hbm, attr_out_hbm, *bufs_and_sems):
    in_bufs = bufs_and_sems[:_NBUF]
    out_bufs = bufs_and_sems[_NBUF:2 * _NBUF]
    at_bufs = bufs_and_sems[2 * _NBUF:3 * _NBUF]
    sems = bufs_and_sems[3 * _NBUF:]
    sins = sems[:_NBUF]
    souts = sems[_NBUF:2 * _NBUF]
    sais = sems[2 * _NBUF:3 * _NBUF]
    saos = sems[3 * _NBUF:]

    cid = lax.axis_index("c")
    sid = lax.axis_index("s")
    wid = sid * _NC + cid

    lane = lax.iota(jnp.int32, _L)

    def chunk_id(t):
        return wid + t * _NW

    def valid(t):
        return chunk_id(t) < _N_CHUNK

    def in_pair(t, b):
        return (in_hbm.at[:, pl.ds(chunk_id(t) * _CC, _CC)], in_bufs[b], sins[b])

    def out_pair(t, b):
        return (out_bufs[b], out_hbm.at[:, pl.ds(chunk_id(t) * _OC, _OC)], souts[b])

    def attr_in_pair(t, b):
        return (attr_hbm.at[:, pl.ds(chunk_id(t) * _CC, _CC)], at_bufs[b], sais[b])

    def attr_out_pair(t, b):
        return (at_bufs[b], attr_out_hbm.at[:, pl.ds(chunk_id(t) * _CC, _CC)], saos[b])

    for p in range(_PD):
        @pl.when(valid(p))
        def _(p=p):
            pltpu.async_copy(*in_pair(p, p % _NBUF))
            pltpu.async_copy(*attr_in_pair(p, p % _NBUF))

    def round_body(t, b):
        bp = (b + _PD) % _NBUF

        @pl.when(valid(t + _PD))
        def _():
            @pl.when(t + _PD >= _NBUF)
            def _():
                pltpu.make_async_copy(*out_pair(t + _PD - _NBUF, bp)).wait()
                pltpu.make_async_copy(*attr_out_pair(t + _PD - _NBUF, bp)).wait()
            pltpu.async_copy(*in_pair(t + _PD, bp))
            pltpu.async_copy(*attr_in_pair(t + _PD, bp))

        @pl.when(valid(t))
        def _():
            pltpu.make_async_copy(*attr_in_pair(t, b)).wait()
            pltpu.async_copy(*attr_out_pair(t, b))
            pltpu.make_async_copy(*in_pair(t, b)).wait()
            src = in_bufs[b]
            dst = out_bufs[b]
            for r in range(_ROWS):
                row_idx = jnp.full((_L,), r, jnp.int32)

                @plsc.parallel_loop(0, _OC // _L, unroll=4)
                def _(i):
                    col_idx = (lane + i * _L) * _DIL
                    dst[r, pl.ds(i * _L, _L)] = plsc.load_gather(
                        src, [row_idx, col_idx])

            pltpu.async_copy(*out_pair(t, b))

    def ring_body(tt, carry):
        for b in range(_NBUF):
            round_body(_NBUF * tt + b, b)
        return carry

    lax.fori_loop(0, _ROUNDS // _NBUF, ring_body, 0)

    for t in range(_ROUNDS):
        @pl.when(jnp.logical_and(valid(t), jnp.logical_not(valid(t + _NBUF))))
        def _(t=t):
            pltpu.make_async_copy(*out_pair(t, t % _NBUF)).wait()
            pltpu.make_async_copy(*attr_out_pair(t, t % _NBUF)).wait()


def kernel(edge_index, edge_attr):
    out, attr_out = _dilate(edge_index, edge_attr)
    return out, attr_out
